# baseline (speedup 1.0000x reference)
import jax
import jax.numpy as jnp
from jax.experimental import pallas as pl
from jax.experimental.pallas import tpu as pltpu
from jax.experimental.pallas import tpu_sc as plsc


_GATHER_WINDOW = 128


def _sc_gather_pairs(table_pairs, idx_phys):
    batch = idx_phys.shape[0]
    width = table_pairs.shape[1]
    idx = idx_phys.reshape(1, batch)
    mesh = plsc.VectorSubcoreMesh(core_axis_name="core",
                                  subcore_axis_name="subcore")

    @pl.kernel(
        out_type=jax.ShapeDtypeStruct((batch, width), table_pairs.dtype),
        mesh=mesh,
    )
    def gather_kernel(table_hbm, idx_hbm, out_hbm):
        def body(idx_vmem, out_vmem):
            pltpu.sync_copy(table_hbm.at[idx_vmem.at[0]], out_vmem)

        pltpu.emit_pipeline(
            body,
            grid=(batch // _GATHER_WINDOW,),
            in_specs=[pl.BlockSpec((1, _GATHER_WINDOW),
                                   index_map=lambda i: (0, i))],
            out_specs=[pl.BlockSpec((_GATHER_WINDOW, width),
                                    index_map=lambda i: (i, 0))],
            core_axis_name=("core", "subcore"),
            dimension_semantics=(pltpu.PARALLEL,),
        )(idx_hbm, out_hbm)

    return gather_kernel(table_pairs, idx)


_VOCAB_TILE = 4096


def _tc_matmul_t(Wt, at):
    embed, vocab = Wt.shape
    batch = at.shape[1]

    def mm_kernel(w_ref, a_ref, o_ref):
        w = w_ref[...].astype(jnp.bfloat16)
        av = a_ref[...].astype(jnp.bfloat16)
        o_ref[...] = jax.lax.dot_general(
            w, av, (((0,), (0,)), ((), ())),
            preferred_element_type=jnp.float32)

    return pl.pallas_call(
        mm_kernel,
        grid=(pl.cdiv(vocab, _VOCAB_TILE),),
        in_specs=[
            pl.BlockSpec((embed, _VOCAB_TILE), lambda i: (0, i)),
            pl.BlockSpec((embed, batch), lambda i: (0, 0)),
        ],
        out_specs=pl.BlockSpec((_VOCAB_TILE, batch), lambda i: (i, 0)),
        out_shape=jax.ShapeDtypeStruct((vocab, batch), jnp.float32),
    )(Wt, at)


def kernel(x, emb_table, W):
    vocab, embed = emb_table.shape
    table_pairs = jnp.concatenate([emb_table[0::2], emb_table[1::2]], axis=1)
    pairs = _sc_gather_pairs(table_pairs, (x >> 1).astype(jnp.int32))
    odd = (x & 1).astype(jnp.bool_).reshape(-1, 1)
    at = jnp.where(odd, pairs[:, embed:], pairs[:, :embed]).T
    outT = _tc_matmul_t(W.T, at)
    return outT.T


# speedup vs baseline: 4.8449x; 4.8449x over previous
import jax
import jax.numpy as jnp
from jax.experimental import pallas as pl
from jax.experimental.pallas import tpu as pltpu
from jax.experimental.pallas import tpu_sc as plsc


_GATHER_WINDOW = 128


def _sc_gather_pairs(table_pairs, idx_phys):
    batch = idx_phys.shape[0]
    width = table_pairs.shape[1]
    idx = idx_phys.reshape(1, batch)
    mesh = plsc.VectorSubcoreMesh(core_axis_name="core",
                                  subcore_axis_name="subcore")

    @pl.kernel(
        out_type=jax.ShapeDtypeStruct((batch, width), table_pairs.dtype),
        mesh=mesh,
    )
    def gather_kernel(table_hbm, idx_hbm, out_hbm):
        def body(idx_vmem, out_vmem):
            pltpu.sync_copy(table_hbm.at[idx_vmem.at[0]], out_vmem)

        pltpu.emit_pipeline(
            body,
            grid=(batch // _GATHER_WINDOW,),
            in_specs=[pl.BlockSpec((1, _GATHER_WINDOW),
                                   index_map=lambda i: (0, i))],
            out_specs=[pl.BlockSpec((_GATHER_WINDOW, width),
                                    index_map=lambda i: (i, 0))],
            core_axis_name=("core", "subcore"),
            dimension_semantics=(pltpu.PARALLEL,),
        )(idx_hbm, out_hbm)

    return gather_kernel(table_pairs, idx)


_VOCAB_TILE = 4096


def _tc_matmul_t(Wt, at):
    embed, vocab = Wt.shape
    batch = at.shape[1]

    def mm_kernel(w_ref, a_ref, o_ref):
        w = w_ref[...].astype(jnp.bfloat16)
        av = a_ref[...].astype(jnp.bfloat16)
        o_ref[...] = jax.lax.dot_general(
            w, av, (((0,), (0,)), ((), ())),
            preferred_element_type=jnp.float32)

    return pl.pallas_call(
        mm_kernel,
        grid=(pl.cdiv(vocab, _VOCAB_TILE),),
        in_specs=[
            pl.BlockSpec((embed, _VOCAB_TILE), lambda i: (0, i)),
            pl.BlockSpec((embed, batch), lambda i: (0, 0)),
        ],
        out_specs=pl.BlockSpec((_VOCAB_TILE, batch), lambda i: (i, 0)),
        out_shape=jax.ShapeDtypeStruct((vocab, batch), jnp.float32),
        compiler_params=pltpu.CompilerParams(
            dimension_semantics=("parallel",)),
    )(Wt, at)


def kernel(x, emb_table, W):
    vocab, embed = emb_table.shape
    table_pairs = emb_table.reshape(vocab // 2, 2 * embed)
    pairs = _sc_gather_pairs(table_pairs, (x >> 1).astype(jnp.int32))
    odd = (x & 1).astype(jnp.bool_).reshape(-1, 1)
    at = jnp.where(odd, pairs[:, embed:], pairs[:, :embed]).T
    outT = _tc_matmul_t(W.T, at)
    return outT.T


# speedup vs baseline: 5.2015x; 1.0736x over previous
import jax
import jax.numpy as jnp
from jax.experimental import pallas as pl
from jax.experimental.pallas import tpu as pltpu
from jax.experimental.pallas import tpu_sc as plsc


_GATHER_WINDOW = 128


def _sc_gather_rows(table_rm, idx):
    batch = idx.shape[0]
    width = table_rm.shape[1]
    idx2 = idx.reshape(1, batch)
    mesh = plsc.VectorSubcoreMesh(core_axis_name="core",
                                  subcore_axis_name="subcore")

    @pl.kernel(
        out_type=jax.ShapeDtypeStruct((batch, width), table_rm.dtype),
        mesh=mesh,
    )
    def gather_kernel(table_hbm, idx_hbm, out_hbm):
        def body(idx_vmem, out_vmem):
            pltpu.sync_copy(table_hbm.at[idx_vmem.at[0]], out_vmem)

        pltpu.emit_pipeline(
            body,
            grid=(batch // _GATHER_WINDOW,),
            in_specs=[pl.BlockSpec((1, _GATHER_WINDOW),
                                   index_map=lambda i: (0, i))],
            out_specs=[pl.BlockSpec((_GATHER_WINDOW, width),
                                    index_map=lambda i: (i, 0))],
            core_axis_name=("core", "subcore"),
            dimension_semantics=(pltpu.PARALLEL,),
        )(idx_hbm, out_hbm)

    return gather_kernel(table_rm, idx2)


_VOCAB_TILE = 2048


def _tc_matmul_t(Wt, rows):
    embed, vocab = Wt.shape
    batch = rows.shape[0]

    def mm_kernel(w_ref, a_ref, o_ref):
        w = w_ref[...].astype(jnp.bfloat16)
        a = a_ref[...].astype(jnp.bfloat16)
        o_ref[...] = jax.lax.dot_general(
            w, a, (((0,), (1,)), ((), ())),
            preferred_element_type=jnp.float32)

    return pl.pallas_call(
        mm_kernel,
        grid=(pl.cdiv(vocab, _VOCAB_TILE),),
        in_specs=[
            pl.BlockSpec((embed, _VOCAB_TILE), lambda i: (0, i)),
            pl.BlockSpec((batch, embed), lambda i: (0, 0)),
        ],
        out_specs=pl.BlockSpec((_VOCAB_TILE, batch), lambda i: (i, 0)),
        out_shape=jax.ShapeDtypeStruct((vocab, batch), jnp.float32),
        compiler_params=pltpu.CompilerParams(
            dimension_semantics=("arbitrary",)),
    )(Wt, rows)


_T_TILE = 1024
_HALF = 50176


def _tc_pair_table(Tv):
    embed, vocab = Tv.shape
    nblk = _HALF // _T_TILE

    def tp_kernel(a_ref, b_ref, o_ref):
        o_ref[...] = jnp.concatenate([a_ref[...].T, b_ref[...].T], axis=1)

    return pl.pallas_call(
        tp_kernel,
        grid=(nblk,),
        in_specs=[
            pl.BlockSpec((embed, _T_TILE), lambda i: (0, i)),
            pl.BlockSpec((embed, _T_TILE), lambda i: (0, i + nblk)),
        ],
        out_specs=pl.BlockSpec((_T_TILE, 2 * embed), lambda i: (i, 0)),
        out_shape=jax.ShapeDtypeStruct((_HALF, 2 * embed), jnp.float32),
        compiler_params=pltpu.CompilerParams(
            dimension_semantics=("arbitrary",)),
    )(Tv, Tv)


def kernel(x, emb_table, W):
    vocab, embed = emb_table.shape
    pairs_tab = _tc_pair_table(emb_table.T)
    xi = x.astype(jnp.int32)
    phys = jnp.where(xi >= _HALF, xi - _HALF, xi)
    pairs = _sc_gather_rows(pairs_tab, phys)
    hi = (xi >= _HALF).reshape(-1, 1)
    rows = jnp.where(hi, pairs[:, embed:], pairs[:, :embed])
    outT = _tc_matmul_t(W.T, rows)
    return outT.T


# speedup vs baseline: 5.2956x; 1.0181x over previous
import jax
import jax.numpy as jnp
from jax.experimental import pallas as pl
from jax.experimental.pallas import tpu as pltpu
from jax.experimental.pallas import tpu_sc as plsc


_GATHER_WINDOW = 128


def _sc_gather_rows(table_rm, idx):
    batch = idx.shape[0]
    width = table_rm.shape[1]
    idx2 = idx.reshape(1, batch)
    mesh = plsc.VectorSubcoreMesh(core_axis_name="core",
                                  subcore_axis_name="subcore")

    @pl.kernel(
        out_type=jax.ShapeDtypeStruct((batch, width), table_rm.dtype),
        mesh=mesh,
    )
    def gather_kernel(table_hbm, idx_hbm, out_hbm):
        def body(idx_vmem, out_vmem):
            pltpu.sync_copy(table_hbm.at[idx_vmem.at[0]], out_vmem)

        pltpu.emit_pipeline(
            body,
            grid=(batch // _GATHER_WINDOW,),
            in_specs=[pl.BlockSpec((1, _GATHER_WINDOW),
                                   index_map=lambda i: (0, i))],
            out_specs=[pl.BlockSpec((_GATHER_WINDOW, width),
                                    index_map=lambda i: (i, 0))],
            core_axis_name=("core", "subcore"),
            dimension_semantics=(pltpu.PARALLEL,),
        )(idx_hbm, out_hbm)

    return gather_kernel(table_rm, idx2)


_VOCAB_TILE = 4096


def _tc_matmul_t(Wt, pairs, hi):
    embed2, vocab = Wt.shape[0] * 2, Wt.shape[1]
    embed = Wt.shape[0]
    batch = pairs.shape[0]

    def mm_kernel(w_ref, p_ref, h_ref, o_ref):
        sel = h_ref[...] != 0
        a = jnp.where(sel, p_ref[:, embed:], p_ref[:, :embed])
        w = w_ref[...].astype(jnp.bfloat16)
        ab = a.astype(jnp.bfloat16)
        o_ref[...] = jax.lax.dot_general(
            w, ab, (((0,), (1,)), ((), ())),
            preferred_element_type=jnp.float32)

    return pl.pallas_call(
        mm_kernel,
        grid=(pl.cdiv(vocab, _VOCAB_TILE),),
        in_specs=[
            pl.BlockSpec((embed, _VOCAB_TILE), lambda i: (0, i)),
            pl.BlockSpec((batch, 2 * embed), lambda i: (0, 0)),
            pl.BlockSpec((batch, 1), lambda i: (0, 0)),
        ],
        out_specs=pl.BlockSpec((_VOCAB_TILE, batch), lambda i: (i, 0)),
        out_shape=jax.ShapeDtypeStruct((vocab, batch), jnp.float32),
        compiler_params=pltpu.CompilerParams(
            dimension_semantics=("parallel",)),
    )(Wt, pairs, hi)


_T_TILE = 1024
_HALF = 50176


def _tc_pair_table(Tv):
    embed, vocab = Tv.shape
    nblk = _HALF // _T_TILE

    def tp_kernel(a_ref, b_ref, o_ref):
        o_ref[...] = jnp.concatenate([a_ref[...].T, b_ref[...].T], axis=1)

    return pl.pallas_call(
        tp_kernel,
        grid=(nblk,),
        in_specs=[
            pl.BlockSpec((embed, _T_TILE), lambda i: (0, i)),
            pl.BlockSpec((embed, _T_TILE), lambda i: (0, i + nblk)),
        ],
        out_specs=pl.BlockSpec((_T_TILE, 2 * embed), lambda i: (i, 0)),
        out_shape=jax.ShapeDtypeStruct((_HALF, 2 * embed), jnp.float32),
        compiler_params=pltpu.CompilerParams(
            dimension_semantics=("arbitrary",)),
    )(Tv, Tv)


def kernel(x, emb_table, W):
    vocab, embed = emb_table.shape
    pairs_tab = _tc_pair_table(emb_table.T)
    xi = x.astype(jnp.int32)
    phys = jnp.where(xi >= _HALF, xi - _HALF, xi)
    pairs = _sc_gather_rows(pairs_tab, phys)
    hi = (xi >= _HALF).astype(jnp.int32).reshape(-1, 1)
    outT = _tc_matmul_t(W.T, pairs, hi)
    return outT.T


# speedup vs baseline: 5.3602x; 1.0122x over previous
import jax
import jax.numpy as jnp
from jax.experimental import pallas as pl
from jax.experimental.pallas import tpu as pltpu
from jax.experimental.pallas import tpu_sc as plsc


_GATHER_WINDOW = 128


def _sc_gather_rows(table_rm, idx):
    batch = idx.shape[0]
    width = table_rm.shape[1]
    idx2 = idx.reshape(1, batch)
    mesh = plsc.VectorSubcoreMesh(core_axis_name="core",
                                  subcore_axis_name="subcore")

    @pl.kernel(
        out_type=jax.ShapeDtypeStruct((batch, width), table_rm.dtype),
        mesh=mesh,
    )
    def gather_kernel(table_hbm, idx_hbm, out_hbm):
        def body(idx_vmem, out_vmem):
            pltpu.sync_copy(table_hbm.at[idx_vmem.at[0]], out_vmem)

        pltpu.emit_pipeline(
            body,
            grid=(batch // _GATHER_WINDOW,),
            in_specs=[pl.BlockSpec((1, _GATHER_WINDOW),
                                   index_map=lambda i: (0, i))],
            out_specs=[pl.BlockSpec((_GATHER_WINDOW, width),
                                    index_map=lambda i: (i, 0))],
            core_axis_name=("core", "subcore"),
            dimension_semantics=(pltpu.PARALLEL,),
        )(idx_hbm, out_hbm)

    return gather_kernel(table_rm, idx2)


_VOCAB_TILE = 4096


def _tc_matmul_t(Wt, pairs, hi):
    embed2, vocab = Wt.shape[0] * 2, Wt.shape[1]
    embed = Wt.shape[0]
    batch = pairs.shape[0]

    def mm_kernel(w_ref, p_ref, h_ref, o_ref):
        sel = h_ref[...] != 0
        ab = jnp.where(sel, p_ref[:, embed:], p_ref[:, :embed])
        w = w_ref[...].astype(jnp.bfloat16)
        o_ref[...] = jax.lax.dot_general(
            w, ab, (((0,), (1,)), ((), ())),
            preferred_element_type=jnp.float32)

    return pl.pallas_call(
        mm_kernel,
        grid=(pl.cdiv(vocab, _VOCAB_TILE),),
        in_specs=[
            pl.BlockSpec((embed, _VOCAB_TILE), lambda i: (0, i)),
            pl.BlockSpec((batch, 2 * embed), lambda i: (0, 0)),
            pl.BlockSpec((batch, 1), lambda i: (0, 0)),
        ],
        out_specs=pl.BlockSpec((_VOCAB_TILE, batch), lambda i: (i, 0)),
        out_shape=jax.ShapeDtypeStruct((vocab, batch), jnp.float32),
        compiler_params=pltpu.CompilerParams(
            dimension_semantics=("parallel",)),
    )(Wt, pairs, hi)


_T_TILE = 1024
_HALF = 50176


def _tc_pair_table(Tv):
    embed, vocab = Tv.shape
    nblk = _HALF // _T_TILE

    def tp_kernel(a_ref, b_ref, o_ref):
        ident = (jax.lax.broadcasted_iota(jnp.int32, (embed, embed), 0)
                 == jax.lax.broadcasted_iota(jnp.int32, (embed, embed), 1)
                 ).astype(jnp.bfloat16)
        at = jax.lax.dot_general(
            a_ref[...].astype(jnp.bfloat16), ident,
            (((0,), (0,)), ((), ())), preferred_element_type=jnp.float32)
        bt = jax.lax.dot_general(
            b_ref[...].astype(jnp.bfloat16), ident,
            (((0,), (0,)), ((), ())), preferred_element_type=jnp.float32)
        o_ref[...] = jnp.concatenate([at, bt], axis=1)

    return pl.pallas_call(
        tp_kernel,
        grid=(nblk,),
        in_specs=[
            pl.BlockSpec((embed, _T_TILE), lambda i: (0, i)),
            pl.BlockSpec((embed, _T_TILE), lambda i: (0, i + nblk)),
        ],
        out_specs=pl.BlockSpec((_T_TILE, 2 * embed), lambda i: (i, 0)),
        out_shape=jax.ShapeDtypeStruct((_HALF, 2 * embed), jnp.float32),
        compiler_params=pltpu.CompilerParams(
            dimension_semantics=("arbitrary",)),
    )(Tv, Tv)


def kernel(x, emb_table, W):
    vocab, embed = emb_table.shape
    pairs_tab = _tc_pair_table(emb_table.T)
    xi = x.astype(jnp.int32)
    phys = jnp.where(xi >= _HALF, xi - _HALF, xi)
    pairs = _sc_gather_rows(pairs_tab, phys)
    hi = (xi >= _HALF).astype(jnp.int32).reshape(-1, 1)
    outT = _tc_matmul_t(W.T, pairs, hi)
    return outT.T


# speedup vs baseline: 5.9062x; 1.1019x over previous
import jax
import jax.numpy as jnp
from jax.experimental import pallas as pl
from jax.experimental.pallas import tpu as pltpu
from jax.experimental.pallas import tpu_sc as plsc


_GATHER_WINDOW = 128


def _sc_gather_rows(table_rm, idx):
    batch = idx.shape[0]
    width = table_rm.shape[1]
    idx2 = idx.reshape(1, batch)
    mesh = plsc.VectorSubcoreMesh(core_axis_name="core",
                                  subcore_axis_name="subcore")

    @pl.kernel(
        out_type=jax.ShapeDtypeStruct((batch, width), table_rm.dtype),
        mesh=mesh,
    )
    def gather_kernel(table_hbm, idx_hbm, out_hbm):
        def body(idx_vmem, out_vmem):
            pltpu.sync_copy(table_hbm.at[idx_vmem.at[0]], out_vmem)

        pltpu.emit_pipeline(
            body,
            grid=(batch // _GATHER_WINDOW,),
            in_specs=[pl.BlockSpec((1, _GATHER_WINDOW),
                                   index_map=lambda i: (0, i))],
            out_specs=[pl.BlockSpec((_GATHER_WINDOW, width),
                                    index_map=lambda i: (i, 0))],
            core_axis_name=("core", "subcore"),
            dimension_semantics=(pltpu.PARALLEL,),
        )(idx_hbm, out_hbm)

    return gather_kernel(table_rm, idx2)


_VOCAB_TILE = 4096


def _tc_matmul_t(Wt, pairs, hi):
    embed2, vocab = Wt.shape[0] * 2, Wt.shape[1]
    embed = Wt.shape[0]
    batch = pairs.shape[0]

    def mm_kernel(w_ref, p_ref, h_ref, o_ref):
        sel = h_ref[...] != 0
        ab = jnp.where(sel, p_ref[:, embed:], p_ref[:, :embed])
        w = w_ref[...].astype(jnp.bfloat16)
        o_ref[...] = jax.lax.dot_general(
            w, ab, (((0,), (1,)), ((), ())),
            preferred_element_type=jnp.float32)

    return pl.pallas_call(
        mm_kernel,
        grid=(pl.cdiv(vocab, _VOCAB_TILE),),
        in_specs=[
            pl.BlockSpec((embed, _VOCAB_TILE), lambda i: (0, i)),
            pl.BlockSpec((batch, 2 * embed), lambda i: (0, 0)),
            pl.BlockSpec((batch, 1), lambda i: (0, 0)),
        ],
        out_specs=pl.BlockSpec((_VOCAB_TILE, batch), lambda i: (i, 0)),
        out_shape=jax.ShapeDtypeStruct((vocab, batch), jnp.float32),
        compiler_params=pltpu.CompilerParams(
            dimension_semantics=("parallel",)),
    )(Wt, pairs, hi)


_T_TILE = 4096
_HALF = 53248


def _tc_pair_table(Tv):
    embed, vocab = Tv.shape
    nblk = _HALF // _T_TILE
    max_b = (vocab - 1) // _T_TILE

    def tp_kernel(a_ref, b_ref, o_ref):
        ident = (jax.lax.broadcasted_iota(jnp.int32, (embed, embed), 0)
                 == jax.lax.broadcasted_iota(jnp.int32, (embed, embed), 1)
                 ).astype(jnp.bfloat16)
        at = jax.lax.dot_general(
            a_ref[...].astype(jnp.bfloat16), ident,
            (((0,), (0,)), ((), ())), preferred_element_type=jnp.float32)
        bt = jax.lax.dot_general(
            b_ref[...].astype(jnp.bfloat16), ident,
            (((0,), (0,)), ((), ())), preferred_element_type=jnp.float32)
        o_ref[...] = jnp.concatenate([at, bt], axis=1)

    return pl.pallas_call(
        tp_kernel,
        grid=(nblk,),
        in_specs=[
            pl.BlockSpec((embed, _T_TILE), lambda i: (0, i)),
            pl.BlockSpec((embed, _T_TILE),
                         lambda i: (0, jnp.minimum(i + nblk, max_b))),
        ],
        out_specs=pl.BlockSpec((_T_TILE, 2 * embed), lambda i: (i, 0)),
        out_shape=jax.ShapeDtypeStruct((_HALF, 2 * embed), jnp.float32),
        compiler_params=pltpu.CompilerParams(
            dimension_semantics=("arbitrary",)),
    )(Tv, Tv)


def kernel(x, emb_table, W):
    vocab, embed = emb_table.shape
    pairs_tab = _tc_pair_table(emb_table.T)
    xi = x.astype(jnp.int32)
    phys = jnp.where(xi >= _HALF, xi - _HALF, xi)
    pairs = _sc_gather_rows(pairs_tab, phys)
    hi = (xi >= _HALF).astype(jnp.int32).reshape(-1, 1)
    outT = _tc_matmul_t(W.T, pairs, hi)
    return outT.T


# speedup vs baseline: 5.9639x; 1.0098x over previous
import jax
import jax.numpy as jnp
from jax.experimental import pallas as pl
from jax.experimental.pallas import tpu as pltpu
from jax.experimental.pallas import tpu_sc as plsc


_GATHER_WINDOW = 128


def _sc_gather_rows(table_rm, idx):
    batch = idx.shape[0]
    width = table_rm.shape[1]
    idx2 = idx.reshape(1, batch)
    mesh = plsc.VectorSubcoreMesh(core_axis_name="core",
                                  subcore_axis_name="subcore")

    @pl.kernel(
        out_type=jax.ShapeDtypeStruct((batch, width), table_rm.dtype),
        mesh=mesh,
    )
    def gather_kernel(table_hbm, idx_hbm, out_hbm):
        def body(idx_vmem, out_vmem):
            pltpu.sync_copy(table_hbm.at[idx_vmem.at[0]], out_vmem)

        pltpu.emit_pipeline(
            body,
            grid=(batch // _GATHER_WINDOW,),
            in_specs=[pl.BlockSpec((1, _GATHER_WINDOW),
                                   index_map=lambda i: (0, i))],
            out_specs=[pl.BlockSpec((_GATHER_WINDOW, width),
                                    index_map=lambda i: (i, 0))],
            core_axis_name=("core", "subcore"),
            dimension_semantics=(pltpu.PARALLEL,),
        )(idx_hbm, out_hbm)

    return gather_kernel(table_rm, idx2)


_VOCAB_TILE = 4096


def _tc_matmul_t(Wt, pairs, hi):
    embed2, vocab = Wt.shape[0] * 2, Wt.shape[1]
    embed = Wt.shape[0]
    batch = pairs.shape[0]

    def mm_kernel(w_ref, p_ref, h_ref, o_ref):
        sel = h_ref[...] != 0
        ab = jnp.where(sel, p_ref[:, embed:], p_ref[:, :embed])
        w = w_ref[...].astype(jnp.bfloat16)
        o_ref[...] = jax.lax.dot_general(
            w, ab, (((0,), (1,)), ((), ())),
            preferred_element_type=jnp.float32)

    return pl.pallas_call(
        mm_kernel,
        grid=(pl.cdiv(vocab, _VOCAB_TILE),),
        in_specs=[
            pl.BlockSpec((embed, _VOCAB_TILE), lambda i: (0, i)),
            pl.BlockSpec((batch, 2 * embed), lambda i: (0, 0)),
            pl.BlockSpec((batch, 1), lambda i: (0, 0)),
        ],
        out_specs=pl.BlockSpec((_VOCAB_TILE, batch), lambda i: (i, 0)),
        out_shape=jax.ShapeDtypeStruct((vocab, batch), jnp.float32),
        compiler_params=pltpu.CompilerParams(
            dimension_semantics=("parallel",)),
    )(Wt, pairs, hi)


_T_TILE = 8192
_HALF = 57344


def _tc_pair_table(Tv):
    embed, vocab = Tv.shape
    nblk = _HALF // _T_TILE
    max_b = (vocab - 1) // _T_TILE

    def tp_kernel(a_ref, b_ref, o_ref):
        ident = (jax.lax.broadcasted_iota(jnp.int32, (embed, embed), 0)
                 == jax.lax.broadcasted_iota(jnp.int32, (embed, embed), 1)
                 ).astype(jnp.bfloat16)
        at = jax.lax.dot_general(
            a_ref[...].astype(jnp.bfloat16), ident,
            (((0,), (0,)), ((), ())), preferred_element_type=jnp.float32)
        bt = jax.lax.dot_general(
            b_ref[...].astype(jnp.bfloat16), ident,
            (((0,), (0,)), ((), ())), preferred_element_type=jnp.float32)
        o_ref[...] = jnp.concatenate([at, bt], axis=1)

    return pl.pallas_call(
        tp_kernel,
        grid=(nblk,),
        in_specs=[
            pl.BlockSpec((embed, _T_TILE), lambda i: (0, i)),
            pl.BlockSpec((embed, _T_TILE),
                         lambda i: (0, jnp.minimum(i + nblk, max_b))),
        ],
        out_specs=pl.BlockSpec((_T_TILE, 2 * embed), lambda i: (i, 0)),
        out_shape=jax.ShapeDtypeStruct((_HALF, 2 * embed), jnp.float32),
        compiler_params=pltpu.CompilerParams(
            dimension_semantics=("arbitrary",)),
    )(Tv, Tv)


def kernel(x, emb_table, W):
    vocab, embed = emb_table.shape
    pairs_tab = _tc_pair_table(emb_table.T)
    xi = x.astype(jnp.int32)
    phys = jnp.where(xi >= _HALF, xi - _HALF, xi)
    pairs = _sc_gather_rows(pairs_tab, phys)
    hi = (xi >= _HALF).astype(jnp.int32).reshape(-1, 1)
    outT = _tc_matmul_t(W.T, pairs, hi)
    return outT.T
